# baseline (speedup 1.0000x reference)
import jax
import jax.numpy as jnp
from jax.experimental import pallas as pl
from jax.experimental.pallas import tpu as pltpu

_ROWS = 16384
_COLS = 4096
_BM = 512


def _minmax_body(x_ref, mn_ref, mx_ref):
    i = pl.program_id(0)
    bmn = jnp.min(x_ref[...])
    bmx = jnp.max(x_ref[...])

    @pl.when(i == 0)
    def _init():
        mn_ref[0, 0] = bmn
        mx_ref[0, 0] = bmx

    @pl.when(i > 0)
    def _acc():
        mn_ref[0, 0] = jnp.minimum(mn_ref[0, 0], bmn)
        mx_ref[0, 0] = jnp.maximum(mx_ref[0, 0], bmx)


def _rescale_body(mn_ref, mx_ref, x_ref, o_ref):
    scale = 1.0 / (mx_ref[0, 0] - mn_ref[0, 0])
    o_ref[...] = (x_ref[...] - mn_ref[0, 0]) * scale


def kernel(inp):
    nblk = _ROWS // _BM
    mn, mx = pl.pallas_call(
        _minmax_body,
        grid=(nblk,),
        in_specs=[pl.BlockSpec((_BM, _COLS), lambda i: (i, 0))],
        out_specs=[
            pl.BlockSpec((1, 1), lambda i: (0, 0), memory_space=pltpu.SMEM),
            pl.BlockSpec((1, 1), lambda i: (0, 0), memory_space=pltpu.SMEM),
        ],
        out_shape=[
            jax.ShapeDtypeStruct((1, 1), jnp.float32),
            jax.ShapeDtypeStruct((1, 1), jnp.float32),
        ],
    )(inp)

    out = pl.pallas_call(
        _rescale_body,
        grid=(nblk,),
        in_specs=[
            pl.BlockSpec(memory_space=pltpu.SMEM),
            pl.BlockSpec(memory_space=pltpu.SMEM),
            pl.BlockSpec((_BM, _COLS), lambda i: (i, 0)),
        ],
        out_specs=pl.BlockSpec((_BM, _COLS), lambda i: (i, 0)),
        out_shape=jax.ShapeDtypeStruct((_ROWS, _COLS), jnp.float32),
    )(mn, mx, inp)
    return out


# speedup vs baseline: 1.0318x; 1.0318x over previous
import jax
import jax.numpy as jnp
from jax.experimental import pallas as pl
from jax.experimental.pallas import tpu as pltpu

_ROWS = 16384
_COLS = 4096
_BM1 = 1024
_BM2 = 512


def _minmax_body(x_ref, mn_ref, mx_ref):
    i = pl.program_id(0)
    bmn = jnp.min(x_ref[...])
    bmx = jnp.max(x_ref[...])

    @pl.when(i == 0)
    def _init():
        mn_ref[0, 0] = bmn
        mx_ref[0, 0] = bmx

    @pl.when(i > 0)
    def _acc():
        mn_ref[0, 0] = jnp.minimum(mn_ref[0, 0], bmn)
        mx_ref[0, 0] = jnp.maximum(mx_ref[0, 0], bmx)


def _rescale_body(mn_ref, mx_ref, x_ref, o_ref):
    scale = 1.0 / (mx_ref[0, 0] - mn_ref[0, 0])
    o_ref[...] = (x_ref[...] - mn_ref[0, 0]) * scale


def kernel(inp):
    mn, mx = pl.pallas_call(
        _minmax_body,
        grid=(_ROWS // _BM1,),
        in_specs=[pl.BlockSpec((_BM1, _COLS), lambda i: (i, 0))],
        out_specs=[
            pl.BlockSpec((1, 1), lambda i: (0, 0), memory_space=pltpu.SMEM),
            pl.BlockSpec((1, 1), lambda i: (0, 0), memory_space=pltpu.SMEM),
        ],
        out_shape=[
            jax.ShapeDtypeStruct((1, 1), jnp.float32),
            jax.ShapeDtypeStruct((1, 1), jnp.float32),
        ],
    )(inp)

    out = pl.pallas_call(
        _rescale_body,
        grid=(_ROWS // _BM2,),
        in_specs=[
            pl.BlockSpec(memory_space=pltpu.SMEM),
            pl.BlockSpec(memory_space=pltpu.SMEM),
            pl.BlockSpec((_BM2, _COLS), lambda i: (i, 0)),
        ],
        out_specs=pl.BlockSpec((_BM2, _COLS), lambda i: (i, 0)),
        out_shape=jax.ShapeDtypeStruct((_ROWS, _COLS), jnp.float32),
    )(mn, mx, inp)
    return out
